# baseline (speedup 1.0000x reference)
import functools

import jax
import jax.numpy as jnp
from jax import lax
from jax.experimental import pallas as pl
from jax.experimental.pallas import tpu as pltpu
from jax.experimental.pallas import tpu_sc as plsc

N_TOK = 16384
N_EXP = 16
LANES = 16
NUM_CORES = 2
NUM_SUBCORES = 16
NUM_WORKERS = NUM_CORES * NUM_SUBCORES
TOK_PER_W = N_TOK // NUM_WORKERS
UNROLL = 2

_mesh = plsc.VectorSubcoreMesh(
    core_axis_name="c", subcore_axis_name="s",
    num_cores=NUM_CORES, num_subcores=NUM_SUBCORES)


@functools.partial(
    pl.kernel,
    out_type=jax.ShapeDtypeStruct((N_EXP, N_TOK), jnp.int32),
    mesh=_mesh,
    scratch_types=[
        pltpu.VMEM((N_EXP, TOK_PER_W), jnp.float32),
        pltpu.VMEM((N_EXP, TOK_PER_W), jnp.int32),
    ],
    compiler_params=pltpu.CompilerParams(needs_layout_passes=False),
)
def _threshold_kernel(st_hbm, ot_hbm, s_v, o_v):
    wid = lax.axis_index("s") * NUM_CORES + lax.axis_index("c")
    t0 = wid * TOK_PER_W
    pltpu.sync_copy(st_hbm.at[:, pl.ds(t0, TOK_PER_W)], s_v)

    one = jnp.ones((LANES,), jnp.int32)
    zero = jnp.zeros((LANES,), jnp.int32)

    def tok_block(t):
        h0 = None
        cnt = None
        for e in range(N_EXP):
            v = s_v[e, pl.ds(t, LANES)]
            h = jnp.where(v > 0.0, one, zero)
            cnt = h if cnt is None else cnt + h
            if e == 0:
                h0 = h
            else:
                o_v[e, pl.ds(t, LANES)] = h
        o_v[0, pl.ds(t, LANES)] = h0 + jnp.where(cnt == zero, one, zero)

    def loop_body(j, carry):
        for u in range(UNROLL):
            tok_block((j * UNROLL + u) * LANES)
        return carry

    lax.fori_loop(0, TOK_PER_W // LANES // UNROLL, loop_body, 0)

    pltpu.sync_copy(o_v, ot_hbm.at[:, pl.ds(t0, TOK_PER_W)])


@jax.jit
def kernel(score):
    return _threshold_kernel(score.T).T


# speedup vs baseline: 1.0173x; 1.0173x over previous
import functools

import jax
import jax.numpy as jnp
from jax import lax
from jax.experimental import pallas as pl
from jax.experimental.pallas import tpu as pltpu
from jax.experimental.pallas import tpu_sc as plsc

N_TOK = 16384
N_EXP = 16
LANES = 16
NUM_CORES = 2
NUM_SUBCORES = 16
NUM_WORKERS = NUM_CORES * NUM_SUBCORES
TOK_PER_W = N_TOK // NUM_WORKERS
CHUNK = TOK_PER_W // 2

_mesh = plsc.VectorSubcoreMesh(
    core_axis_name="c", subcore_axis_name="s",
    num_cores=NUM_CORES, num_subcores=NUM_SUBCORES)


@functools.partial(
    pl.kernel,
    out_type=jax.ShapeDtypeStruct((N_EXP, N_TOK), jnp.int32),
    mesh=_mesh,
    scratch_types=[
        pltpu.VMEM((2, N_EXP, CHUNK), jnp.float32),
        pltpu.VMEM((2, N_EXP, CHUNK), jnp.int32),
        pltpu.SemaphoreType.DMA,
        pltpu.SemaphoreType.DMA,
        pltpu.SemaphoreType.DMA,
    ],
)
def _threshold_kernel(st_hbm, ot_hbm, s_v, o_v, sem0, sem1, sem_out):
    wid = lax.axis_index("s") * NUM_CORES + lax.axis_index("c")
    t0 = wid * TOK_PER_W

    one = jnp.ones((LANES,), jnp.int32)
    zero = jnp.zeros((LANES,), jnp.int32)

    def compute(buf):
        for j in range(CHUNK // LANES):
            t = j * LANES
            h0 = None
            cnt = None
            for e in range(N_EXP):
                v = s_v[buf, e, pl.ds(t, LANES)]
                h = jnp.where(v > 0.0, one, zero)
                cnt = h if cnt is None else cnt + h
                if e == 0:
                    h0 = h
                else:
                    o_v[buf, e, pl.ds(t, LANES)] = h
            o_v[buf, 0, pl.ds(t, LANES)] = h0 + jnp.where(cnt == zero, one, zero)

    in0 = pltpu.async_copy(
        st_hbm.at[:, pl.ds(t0, CHUNK)], s_v.at[0], sem0)
    in1 = pltpu.async_copy(
        st_hbm.at[:, pl.ds(t0 + CHUNK, CHUNK)], s_v.at[1], sem1)
    in0.wait()
    compute(0)
    out0 = pltpu.async_copy(
        o_v.at[0], ot_hbm.at[:, pl.ds(t0, CHUNK)], sem_out)
    in1.wait()
    compute(1)
    out1 = pltpu.async_copy(
        o_v.at[1], ot_hbm.at[:, pl.ds(t0 + CHUNK, CHUNK)], sem_out)
    out0.wait()
    out1.wait()


@jax.jit
def kernel(score):
    return _threshold_kernel(score.T).T
